# baseline (speedup 1.0000x reference)
import jax
import jax.numpy as jnp
from jax import lax
from jax.experimental import pallas as pl
from jax.experimental.pallas import tpu as pltpu
from jax.experimental.pallas import tpu_sc as plsc

N = 10000
NP = 10240
E = 320000
NTILES = 16
PT = NP // NTILES
CR = 32
NCHN = PT // CR
CED = 128
NCH = 160
E2 = NTILES * NCH * CED
NTRASH = 10000


def _bcast(ref, idx):
    return plsc.load_gather(ref, [jnp.zeros((16,), jnp.int32) + idx])


def _make_sc_layer(width, prep):
    mesh = plsc.VectorSubcoreMesh(core_axis_name="c", subcore_axis_name="s")

    if prep:
        out_type = (
            jax.ShapeDtypeStruct((3, 2, NP, width), jnp.float32),
            jax.ShapeDtypeStruct((2, NP, width), jnp.float32),
            jax.ShapeDtypeStruct((NP,), jnp.float32),
        )
    else:
        out_type = (
            jax.ShapeDtypeStruct((3, 2, NP, width), jnp.float32),
            jax.ShapeDtypeStruct((2, NP, width), jnp.float32),
        )

    scratch = [
        pltpu.VMEM_SHARED((NP, width), jnp.float32),
        pltpu.VMEM((2, CED), jnp.int32),
        pltpu.VMEM((2, CED), jnp.int32),
        pltpu.VMEM((2, CED, width), jnp.float32),
        pltpu.VMEM((CR, width), jnp.float32),
        pltpu.VMEM((CR, width), jnp.float32),
        pltpu.VMEM((CR, width), jnp.float32),
        pltpu.VMEM((CR, width), jnp.float32),
        pltpu.VMEM((CR, width), jnp.float32),
        pltpu.VMEM((PT,), jnp.float32),
        pltpu.VMEM((CED,), jnp.float32),
        pltpu.SemaphoreType.DMA,
        pltpu.SemaphoreType.DMA,
    ]
    if prep:
        scratch.insert(1, pltpu.VMEM_SHARED((NP,), jnp.float32))

    def body(*refs):
        if prep:
            (h_hbm, row_hbm, col_hbm, tx_hbm, tp_hbm, dinv_hbm,
             acc, deg, ridx, cidx, gbuf,
             abuf, pbuf, tbuf, sbuf, zbuf, dbuf, vbuf, gsem, isem) = refs
        else:
            (h_hbm, row_hbm, col_hbm, dinv_in_hbm, tx_hbm, tp_hbm,
             acc, ridx, cidx, gbuf,
             abuf, pbuf, tbuf, sbuf, zbuf, dbuf, vbuf, gsem, isem) = refs

        cid = lax.axis_index("c")
        sid = lax.axis_index("s")
        r0 = sid * PT
        z16 = jnp.zeros((16,), jnp.float32)
        tpc = tp_hbm.at[cid]

        def zero_rows(ref, nrows):
            def zr(j, _):
                for v in range(width // 16):
                    ref[j, pl.ds(16 * v, 16)] = z16
                return 0
            lax.fori_loop(0, nrows, zr, 0)

        zero_rows(zbuf, CR)

        def fire_idx(i, b):
            pltpu.async_copy(row_hbm.at[sid, i], ridx.at[b], isem)
            pltpu.async_copy(col_hbm.at[sid, i], cidx.at[b], isem)

        def wait_idx():
            pltpu.make_async_copy(
                row_hbm.at[sid, 0], ridx.at[0], isem).wait()
            pltpu.make_async_copy(
                col_hbm.at[sid, 0], cidx.at[0], isem).wait()

        def load_idx_sync(i, b):
            pltpu.sync_copy(row_hbm.at[sid, i], ridx.at[b])
            pltpu.sync_copy(col_hbm.at[sid, i], cidx.at[b])

        def remap(b):
            for v in range(CED // 16):
                r = ridx[b, pl.ds(16 * v, 16)]
                c = cidx[b, pl.ds(16 * v, 16)]
                ridx[b, pl.ds(16 * v, 16)] = jnp.where(r == c, NTRASH, r)

        if prep:
            def zd(j, _):
                dbuf[pl.ds(16 * j, 16)] = z16
                return 0
            lax.fori_loop(0, PT // 16, zd, 0)
            pltpu.sync_copy(dbuf, deg.at[pl.ds(r0, PT)])

            one = jnp.ones((16,), jnp.float32)

            def ob(j, _):
                vbuf[pl.ds(16 * j, 16)] = one
                return 0
            lax.fori_loop(0, CED // 16, ob, 0)
            plsc.subcore_barrier()

            load_idx_sync(0, 0)
            fire_idx(1, 1)

            def dscat(b):
                pltpu.sync_copy(vbuf, deg.at[ridx.at[b]], add=True)

            def dpair(y, _):
                i0 = 2 * y
                remap(0)
                dscat(0)
                wait_idx()
                fire_idx(i0 + 2, 0)
                remap(1)
                dscat(1)
                wait_idx()
                fire_idx(i0 + 3, 1)
                return 0

            lax.fori_loop(0, NCH // 2 - 1, dpair, 0)
            remap(0)
            dscat(0)
            wait_idx()
            remap(1)
            dscat(1)
            plsc.subcore_barrier()

            pltpu.sync_copy(deg.at[pl.ds(r0, PT)], dbuf)

            def dv(j, _):
                d = dbuf[pl.ds(16 * j, 16)]
                ii = lax.bitcast_convert_type(d, jnp.int32)
                yi = jnp.int32(0x5F3759DF) - (ii >> 1)
                y = lax.bitcast_convert_type(yi, jnp.float32)
                for _ in range(3):
                    y = y * (1.5 - 0.5 * d * y * y)
                dbuf[pl.ds(16 * j, 16)] = jnp.where(d > 0.5, y, 0.0)
                return 0

            lax.fori_loop(0, PT // 16, dv, 0)

            @pl.when(cid == 0)
            def _():
                pltpu.sync_copy(dbuf, dinv_hbm.at[pl.ds(r0, PT)])
        else:
            pltpu.sync_copy(dinv_in_hbm.at[pl.ds(r0, PT)], dbuf)

        def init_chunk(cc, _):
            rr = r0 + cc * CR
            pltpu.sync_copy(h_hbm.at[cid, pl.ds(rr, CR), :], pbuf)

            def tinit(j, _):
                d = _bcast(dbuf, cc * CR + j)
                for v in range(width // 16):
                    sbuf[j, pl.ds(16 * v, 16)] = d * pbuf[j, pl.ds(16 * v, 16)]
                return 0

            lax.fori_loop(0, CR, tinit, 0)
            pltpu.sync_copy(sbuf, tpc.at[pl.ds(rr, CR), :])
            pltpu.sync_copy(zbuf, acc.at[pl.ds(rr, CR), :])
            return 0

        lax.fori_loop(0, NCHN, init_chunk, 0)
        plsc.subcore_barrier()

        def fire_gather(b):
            pltpu.async_copy(tpc.at[ridx.at[b]], gbuf.at[b], gsem)

        def wait_gather():
            pltpu.make_async_copy(
                tpc.at[ridx.at[0]], gbuf.at[0], gsem).wait()

        def scatter(b):
            pltpu.sync_copy(gbuf.at[b], acc.at[cidx.at[b]], add=True)

        for k in (1, 2, 3):
            load_idx_sync(0, 0)
            remap(0)
            fire_gather(0)
            fire_idx(1, 1)

            def pair(y, _):
                i0 = 2 * y
                wait_idx()
                remap(1)
                wait_gather()
                fire_gather(1)
                scatter(0)
                fire_idx(i0 + 2, 0)
                wait_idx()
                remap(0)
                wait_gather()
                fire_gather(0)
                scatter(1)
                fire_idx(i0 + 3, 1)
                return 0

            lax.fori_loop(0, NCH // 2 - 1, pair, 0)
            wait_idx()
            remap(1)
            wait_gather()
            fire_gather(1)
            scatter(0)
            wait_gather()
            scatter(1)
            plsc.subcore_barrier()

            def fix_chunk(cc, _):
                rr = r0 + cc * CR
                pltpu.sync_copy(acc.at[pl.ds(rr, CR), :], abuf)
                if k == 2:
                    pltpu.sync_copy(h_hbm.at[cid, pl.ds(rr, CR), :], pbuf)
                elif k == 3:
                    pltpu.sync_copy(
                        tx_hbm.at[0, cid, pl.ds(rr, CR), :], pbuf)

                def fix(j, _):
                    d = _bcast(dbuf, cc * CR + j)
                    for v in range(width // 16):
                        a = abuf[j, pl.ds(16 * v, 16)]
                        if k == 1:
                            t = (-d) * a
                        else:
                            t = ((-2.0 * d) * a
                                 - pbuf[j, pl.ds(16 * v, 16)])
                        tbuf[j, pl.ds(16 * v, 16)] = t
                        if k < 3:
                            sbuf[j, pl.ds(16 * v, 16)] = d * t
                    return 0

                lax.fori_loop(0, CR, fix, 0)
                pltpu.sync_copy(tbuf, tx_hbm.at[k - 1, cid, pl.ds(rr, CR), :])
                if k < 3:
                    pltpu.sync_copy(sbuf, tpc.at[pl.ds(rr, CR), :])
                    pltpu.sync_copy(zbuf, acc.at[pl.ds(rr, CR), :])
                return 0

            lax.fori_loop(0, NCHN, fix_chunk, 0)
            plsc.subcore_barrier()

    return pl.kernel(body, out_type=out_type, mesh=mesh,
                     scratch_types=scratch,
                     compiler_params=pltpu.CompilerParams(
                         needs_layout_passes=False,
                         use_tc_tiling_on_sc=False))


def _tc_conv0(h0p, txs, wfull, whalf, b0):
    RB = 1024
    grid = (NP // RB,)

    def body(h_ref, t0, t1, t2, t3, t4, t5, wf_ref, wh_ref, b_ref, o_ref):
        acc = jnp.dot(h_ref[...], wf_ref[...],
                      preferred_element_type=jnp.float32)
        for i, t in enumerate((t0, t1, t2, t3, t4, t5)):
            acc += jnp.dot(t[...], wh_ref[i],
                           preferred_element_type=jnp.float32)
        acc += b_ref[...]
        acc = jnp.maximum(acc, 0.0)
        rid = (pl.program_id(0) * RB
               + lax.broadcasted_iota(jnp.int32, (RB, 1), 0))
        o_ref[...] = jnp.where(rid < N, acc, 0.0)

    hw = h0p.shape[1]
    tw = txs[0].shape[1]
    return pl.pallas_call(
        body,
        grid=grid,
        in_specs=[pl.BlockSpec((RB, hw), lambda i: (i, 0))]
        + [pl.BlockSpec((RB, tw), lambda i: (i, 0))] * 6
        + [pl.BlockSpec(wfull.shape, lambda i: (0, 0)),
           pl.BlockSpec(whalf.shape, lambda i: (0, 0, 0)),
           pl.BlockSpec(b0.shape, lambda i: (0, 0))],
        out_specs=pl.BlockSpec((RB, 128), lambda i: (i, 0)),
        out_shape=jax.ShapeDtypeStruct((NP, 128), jnp.float32),
    )(h0p, *txs, wfull, whalf, b0)


def _tc_heads(h1, txs, wfull, whalf, b1, mu_w, mu_b, lv_w, lv_b):
    RB = 1024
    grid = (NP // RB,)

    def body(h_ref, t0, t1, t2, t3, t4, t5, wf_ref, wh_ref, b_ref,
             muw_ref, mub_ref, lvw_ref, lvb_ref, mu_ref, lv_ref):
        acc = jnp.dot(h_ref[...], wf_ref[...],
                      preferred_element_type=jnp.float32)
        for i, t in enumerate((t0, t1, t2, t3, t4, t5)):
            acc += jnp.dot(t[...], wh_ref[i],
                           preferred_element_type=jnp.float32)
        h2 = jnp.maximum(acc + b_ref[...], 0.0)
        mu_ref[...] = jnp.dot(h2, muw_ref[...],
                              preferred_element_type=jnp.float32) + mub_ref[...]
        lv_ref[...] = jnp.dot(h2, lvw_ref[...],
                              preferred_element_type=jnp.float32) + lvb_ref[...]

    return pl.pallas_call(
        body,
        grid=grid,
        in_specs=[pl.BlockSpec((RB, 128), lambda i: (i, 0))]
        + [pl.BlockSpec((RB, 64), lambda i: (i, 0))] * 6
        + [pl.BlockSpec(wfull.shape, lambda i: (0, 0)),
           pl.BlockSpec(whalf.shape, lambda i: (0, 0, 0)),
           pl.BlockSpec(b1.shape, lambda i: (0, 0)),
           pl.BlockSpec(mu_w.shape, lambda i: (0, 0)),
           pl.BlockSpec(mu_b.shape, lambda i: (0, 0)),
           pl.BlockSpec(lv_w.shape, lambda i: (0, 0)),
           pl.BlockSpec(lv_b.shape, lambda i: (0, 0))],
        out_specs=[pl.BlockSpec((RB, 64), lambda i: (i, 0)),
                   pl.BlockSpec((RB, 64), lambda i: (i, 0))],
        out_shape=[jax.ShapeDtypeStruct((NP, 64), jnp.float32),
                   jax.ShapeDtypeStruct((NP, 64), jnp.float32)],
    )(h1, *txs, wfull, whalf, b1, mu_w, mu_b, lv_w, lv_b)


_sc_layer0 = _make_sc_layer(80, prep=True)
_sc_layer1 = _make_sc_layer(64, prep=False)


def kernel(x, edge_index, lap_pe, conv0_W, conv0_b, conv1_W, conv1_b,
           mu_W, mu_b, lv_W, lv_b):
    f32 = jnp.float32
    h0 = jnp.concatenate([x, lap_pe], axis=1)
    h0p = jnp.zeros((NP, 160), f32).at[:N, :144].set(h0)
    h0_halves = h0p.reshape(NP, 2, 80).transpose(1, 0, 2)

    rowp = jnp.concatenate(
        [edge_index[0], jnp.full((E2 - E,), NTRASH, jnp.int32)]
    ).reshape(NTILES, NCH, CED)
    colp = jnp.concatenate(
        [edge_index[1], jnp.zeros((E2 - E,), jnp.int32)]
    ).reshape(NTILES, NCH, CED)

    tx0, _, dinv = _sc_layer0(h0_halves, rowp, colp)

    w0p = jnp.zeros((4, 160, 128), f32).at[:, :144, :].set(conv0_W)
    w0_half = jnp.stack([w0p[k, c * 80:(c + 1) * 80]
                         for k in (1, 2, 3) for c in (0, 1)])
    tx0_list = [tx0[k, c] for k in range(3) for c in range(2)]
    h1 = _tc_conv0(h0p, tx0_list, w0p[0], w0_half,
                   conv0_b.reshape(1, 128))

    h1_halves = h1.reshape(NP, 2, 64).transpose(1, 0, 2)
    tx1, _ = _sc_layer1(h1_halves, rowp, colp, dinv)

    w1_half = jnp.stack([conv1_W[k, c * 64:(c + 1) * 64]
                         for k in (1, 2, 3) for c in (0, 1)])
    tx1_list = [tx1[k, c] for k in range(3) for c in range(2)]
    mu_full, lv_full = _tc_heads(
        h1, tx1_list, conv1_W[0], w1_half, conv1_b.reshape(1, 128),
        mu_W, mu_b.reshape(1, 64), lv_W, lv_b.reshape(1, 64))

    return mu_full[:N], lv_full[:N]


# speedup vs baseline: 2.0824x; 2.0824x over previous
import jax
import jax.numpy as jnp
from jax import lax
from jax.experimental import pallas as pl
from jax.experimental.pallas import tpu as pltpu
from jax.experimental.pallas import tpu_sc as plsc

N = 10000
NP = 10240
E = 320000
NTILES = 16
PT = NP // NTILES
CR = 32
NCHN = PT // CR
CED = 80
NCH = E // (NTILES * CED)
NTRASH = 10000


def _bcast(ref, idx):
    return plsc.load_gather(ref, [jnp.zeros((16,), jnp.int32) + idx])


def _make_sc_layer(width, prep):
    mesh = plsc.VectorSubcoreMesh(core_axis_name="c", subcore_axis_name="s")

    if prep:
        out_type = (
            jax.ShapeDtypeStruct((3, 2, NP, width), jnp.float32),
            jax.ShapeDtypeStruct((2, NP, width), jnp.float32),
            jax.ShapeDtypeStruct((NP,), jnp.float32),
        )
    else:
        out_type = (
            jax.ShapeDtypeStruct((3, 2, NP, width), jnp.float32),
            jax.ShapeDtypeStruct((2, NP, width), jnp.float32),
        )

    scratch = [
        pltpu.VMEM_SHARED((NP, width), jnp.float32),
        pltpu.VMEM((4, CED), jnp.int32),
        pltpu.VMEM((4, CED), jnp.int32),
        pltpu.VMEM((4, CED, width), jnp.float32),
        pltpu.VMEM((CR, width), jnp.float32),
        pltpu.VMEM((CR, width), jnp.float32),
        pltpu.VMEM((CR, width), jnp.float32),
        pltpu.VMEM((CR, width), jnp.float32),
        pltpu.VMEM((CR, width), jnp.float32),
        pltpu.VMEM((PT,), jnp.float32),
        pltpu.VMEM((CED,), jnp.float32),
        pltpu.SemaphoreType.DMA,
        pltpu.SemaphoreType.DMA,
        pltpu.SemaphoreType.DMA,
    ]
    if prep:
        scratch.insert(1, pltpu.VMEM_SHARED((NP,), jnp.float32))

    def body(*refs):
        if prep:
            (h_hbm, row_hbm, col_hbm, tx_hbm, tp_hbm, dinv_hbm,
             acc, deg, ridx, cidx, gbuf,
             abuf, pbuf, tbuf, sbuf, zbuf, dbuf, vbuf,
             gsem, isem, ssem) = refs
        else:
            (h_hbm, row_hbm, col_hbm, dinv_in_hbm, tx_hbm, tp_hbm,
             acc, ridx, cidx, gbuf,
             abuf, pbuf, tbuf, sbuf, zbuf, dbuf, vbuf,
             gsem, isem, ssem) = refs

        cid = lax.axis_index("c")
        sid = lax.axis_index("s")
        r0 = sid * PT
        z16 = jnp.zeros((16,), jnp.float32)
        tpc = tp_hbm.at[cid]

        def zero_rows(ref, nrows):
            def zr(j, _):
                for v in range(width // 16):
                    ref[j, pl.ds(16 * v, 16)] = z16
                return 0
            lax.fori_loop(0, nrows, zr, 0)

        zero_rows(zbuf, CR)

        def fire_idx(i, b):
            pltpu.async_copy(row_hbm.at[sid, i], ridx.at[b], isem)
            pltpu.async_copy(col_hbm.at[sid, i], cidx.at[b], isem)

        def wait_idx():
            pltpu.make_async_copy(
                row_hbm.at[sid, 0], ridx.at[0], isem).wait()
            pltpu.make_async_copy(
                col_hbm.at[sid, 0], cidx.at[0], isem).wait()

        def load_idx_sync(i, b):
            pltpu.sync_copy(row_hbm.at[sid, i], ridx.at[b])
            pltpu.sync_copy(col_hbm.at[sid, i], cidx.at[b])

        def remap(b):
            for v in range(CED // 16):
                r = ridx[b, pl.ds(16 * v, 16)]
                c = cidx[b, pl.ds(16 * v, 16)]
                ridx[b, pl.ds(16 * v, 16)] = jnp.where(r == c, NTRASH, r)

        if prep:
            def zd(j, _):
                dbuf[pl.ds(16 * j, 16)] = z16
                return 0
            lax.fori_loop(0, PT // 16, zd, 0)
            pltpu.sync_copy(dbuf, deg.at[pl.ds(r0, PT)])

            one = jnp.ones((16,), jnp.float32)

            def ob(j, _):
                vbuf[pl.ds(16 * j, 16)] = one
                return 0
            lax.fori_loop(0, CED // 16, ob, 0)
            plsc.subcore_barrier()

            load_idx_sync(0, 0)
            fire_idx(1, 1)

            def dscat(b):
                pltpu.sync_copy(vbuf, deg.at[ridx.at[b]], add=True)

            def dpair(y, _):
                i0 = 2 * y
                remap(0)
                dscat(0)
                wait_idx()
                fire_idx(i0 + 2, 0)
                remap(1)
                dscat(1)
                wait_idx()
                fire_idx(i0 + 3, 1)
                return 0

            lax.fori_loop(0, NCH // 2 - 1, dpair, 0)
            remap(0)
            dscat(0)
            wait_idx()
            remap(1)
            dscat(1)
            plsc.subcore_barrier()

            pltpu.sync_copy(deg.at[pl.ds(r0, PT)], dbuf)

            def dv(j, _):
                d = dbuf[pl.ds(16 * j, 16)]
                ii = lax.bitcast_convert_type(d, jnp.int32)
                yi = jnp.int32(0x5F3759DF) - (ii >> 1)
                y = lax.bitcast_convert_type(yi, jnp.float32)
                for _ in range(3):
                    y = y * (1.5 - 0.5 * d * y * y)
                dbuf[pl.ds(16 * j, 16)] = jnp.where(d > 0.5, y, 0.0)
                return 0

            lax.fori_loop(0, PT // 16, dv, 0)

            @pl.when(cid == 0)
            def _():
                pltpu.sync_copy(dbuf, dinv_hbm.at[pl.ds(r0, PT)])
        else:
            pltpu.sync_copy(dinv_in_hbm.at[pl.ds(r0, PT)], dbuf)

        def init_chunk(cc, _):
            rr = r0 + cc * CR
            pltpu.sync_copy(h_hbm.at[cid, pl.ds(rr, CR), :], pbuf)

            def tinit(j, _):
                d = _bcast(dbuf, cc * CR + j)
                for v in range(width // 16):
                    sbuf[j, pl.ds(16 * v, 16)] = d * pbuf[j, pl.ds(16 * v, 16)]
                return 0

            lax.fori_loop(0, CR, tinit, 0)
            pltpu.sync_copy(sbuf, tpc.at[pl.ds(rr, CR), :])
            pltpu.sync_copy(zbuf, acc.at[pl.ds(rr, CR), :])
            return 0

        lax.fori_loop(0, NCHN, init_chunk, 0)
        plsc.subcore_barrier()

        def fire_gather(b):
            pltpu.async_copy(tpc.at[ridx.at[b]], gbuf.at[b], gsem)

        def wait_gather():
            pltpu.make_async_copy(
                tpc.at[ridx.at[0]], gbuf.at[0], gsem).wait()

        def scatter(b):
            pltpu.sync_copy(gbuf.at[b], acc.at[cidx.at[b]], add=True)

        def fire_scatter(b):
            pltpu.async_copy(gbuf.at[b], acc.at[cidx.at[b]], ssem, add=True)

        def wait_scatter():
            pltpu.make_async_copy(
                gbuf.at[0], acc.at[cidx.at[0]], ssem).wait()

        for k in (1, 2, 3):
            load_idx_sync(0, 0)
            remap(0)
            fire_gather(0)
            fire_idx(1, 1)

            def process(i, s):
                wait_idx()
                remap((s + 1) % 4)
                fire_gather((s + 1) % 4)
                wait_scatter()
                fire_idx(i + 2, (s + 2) % 4)
                wait_gather()
                fire_scatter(s)

            wait_idx()
            remap(1)
            fire_gather(1)
            fire_idx(2, 2)
            wait_gather()
            fire_scatter(0)
            wait_idx()
            remap(2)
            fire_gather(2)
            fire_idx(3, 3)
            wait_gather()
            fire_scatter(1)

            def quad(q, _):
                i = 4 * q + 2
                process(i, 2)
                process(i + 1, 3)
                process(i + 2, 0)
                process(i + 3, 1)
                return 0

            lax.fori_loop(0, (NCH - 6) // 4, quad, 0)
            process(NCH - 4, 2)
            process(NCH - 3, 3)
            wait_idx()
            remap(1)
            fire_gather(1)
            wait_scatter()
            wait_gather()
            fire_scatter(0)
            wait_scatter()
            wait_gather()
            fire_scatter(1)
            wait_scatter()
            wait_scatter()
            plsc.subcore_barrier()

            def fix_chunk(cc, _):
                rr = r0 + cc * CR
                pltpu.sync_copy(acc.at[pl.ds(rr, CR), :], abuf)
                if k == 2:
                    pltpu.sync_copy(h_hbm.at[cid, pl.ds(rr, CR), :], pbuf)
                elif k == 3:
                    pltpu.sync_copy(
                        tx_hbm.at[0, cid, pl.ds(rr, CR), :], pbuf)

                def fix(j, _):
                    d = _bcast(dbuf, cc * CR + j)
                    for v in range(width // 16):
                        a = abuf[j, pl.ds(16 * v, 16)]
                        if k == 1:
                            t = (-d) * a
                        else:
                            t = ((-2.0 * d) * a
                                 - pbuf[j, pl.ds(16 * v, 16)])
                        tbuf[j, pl.ds(16 * v, 16)] = t
                        if k < 3:
                            sbuf[j, pl.ds(16 * v, 16)] = d * t
                    return 0

                lax.fori_loop(0, CR, fix, 0)
                pltpu.sync_copy(tbuf, tx_hbm.at[k - 1, cid, pl.ds(rr, CR), :])
                if k < 3:
                    pltpu.sync_copy(sbuf, tpc.at[pl.ds(rr, CR), :])
                    pltpu.sync_copy(zbuf, acc.at[pl.ds(rr, CR), :])
                return 0

            lax.fori_loop(0, NCHN, fix_chunk, 0)
            plsc.subcore_barrier()

    return pl.kernel(body, out_type=out_type, mesh=mesh,
                     scratch_types=scratch,
                     compiler_params=pltpu.CompilerParams(
                         needs_layout_passes=False,
                         use_tc_tiling_on_sc=False))


def _tc_conv0(h0p, txs, wfull, whalf, b0):
    RB = 1024
    grid = (NP // RB,)

    def body(h_ref, t0, t1, t2, t3, t4, t5, wf_ref, wh_ref, b_ref, o_ref):
        acc = jnp.dot(h_ref[...], wf_ref[...],
                      preferred_element_type=jnp.float32)
        for i, t in enumerate((t0, t1, t2, t3, t4, t5)):
            acc += jnp.dot(t[...], wh_ref[i],
                           preferred_element_type=jnp.float32)
        acc += b_ref[...]
        acc = jnp.maximum(acc, 0.0)
        rid = (pl.program_id(0) * RB
               + lax.broadcasted_iota(jnp.int32, (RB, 1), 0))
        o_ref[...] = jnp.where(rid < N, acc, 0.0)

    hw = h0p.shape[1]
    tw = txs[0].shape[1]
    return pl.pallas_call(
        body,
        grid=grid,
        in_specs=[pl.BlockSpec((RB, hw), lambda i: (i, 0))]
        + [pl.BlockSpec((RB, tw), lambda i: (i, 0))] * 6
        + [pl.BlockSpec(wfull.shape, lambda i: (0, 0)),
           pl.BlockSpec(whalf.shape, lambda i: (0, 0, 0)),
           pl.BlockSpec(b0.shape, lambda i: (0, 0))],
        out_specs=pl.BlockSpec((RB, 128), lambda i: (i, 0)),
        out_shape=jax.ShapeDtypeStruct((NP, 128), jnp.float32),
    )(h0p, *txs, wfull, whalf, b0)


def _tc_heads(h1, txs, wfull, whalf, b1, mu_w, mu_b, lv_w, lv_b):
    RB = 1024
    grid = (NP // RB,)

    def body(h_ref, t0, t1, t2, t3, t4, t5, wf_ref, wh_ref, b_ref,
             muw_ref, mub_ref, lvw_ref, lvb_ref, mu_ref, lv_ref):
        acc = jnp.dot(h_ref[...], wf_ref[...],
                      preferred_element_type=jnp.float32)
        for i, t in enumerate((t0, t1, t2, t3, t4, t5)):
            acc += jnp.dot(t[...], wh_ref[i],
                           preferred_element_type=jnp.float32)
        h2 = jnp.maximum(acc + b_ref[...], 0.0)
        mu_ref[...] = jnp.dot(h2, muw_ref[...],
                              preferred_element_type=jnp.float32) + mub_ref[...]
        lv_ref[...] = jnp.dot(h2, lvw_ref[...],
                              preferred_element_type=jnp.float32) + lvb_ref[...]

    return pl.pallas_call(
        body,
        grid=grid,
        in_specs=[pl.BlockSpec((RB, 128), lambda i: (i, 0))]
        + [pl.BlockSpec((RB, 64), lambda i: (i, 0))] * 6
        + [pl.BlockSpec(wfull.shape, lambda i: (0, 0)),
           pl.BlockSpec(whalf.shape, lambda i: (0, 0, 0)),
           pl.BlockSpec(b1.shape, lambda i: (0, 0)),
           pl.BlockSpec(mu_w.shape, lambda i: (0, 0)),
           pl.BlockSpec(mu_b.shape, lambda i: (0, 0)),
           pl.BlockSpec(lv_w.shape, lambda i: (0, 0)),
           pl.BlockSpec(lv_b.shape, lambda i: (0, 0))],
        out_specs=[pl.BlockSpec((RB, 64), lambda i: (i, 0)),
                   pl.BlockSpec((RB, 64), lambda i: (i, 0))],
        out_shape=[jax.ShapeDtypeStruct((NP, 64), jnp.float32),
                   jax.ShapeDtypeStruct((NP, 64), jnp.float32)],
    )(h1, *txs, wfull, whalf, b1, mu_w, mu_b, lv_w, lv_b)


_sc_layer0 = _make_sc_layer(80, prep=True)
_sc_layer1 = _make_sc_layer(64, prep=False)


def kernel(x, edge_index, lap_pe, conv0_W, conv0_b, conv1_W, conv1_b,
           mu_W, mu_b, lv_W, lv_b):
    f32 = jnp.float32
    h0 = jnp.concatenate([x, lap_pe], axis=1)
    h0p = jnp.zeros((NP, 160), f32).at[:N, :144].set(h0)
    h0_halves = h0p.reshape(NP, 2, 80).transpose(1, 0, 2)

    rowp = edge_index[0].reshape(NTILES, NCH, CED)
    colp = edge_index[1].reshape(NTILES, NCH, CED)

    tx0, _, dinv = _sc_layer0(h0_halves, rowp, colp)

    w0p = jnp.zeros((4, 160, 128), f32).at[:, :144, :].set(conv0_W)
    w0_half = jnp.stack([w0p[k, c * 80:(c + 1) * 80]
                         for k in (1, 2, 3) for c in (0, 1)])
    tx0_list = [tx0[k, c] for k in range(3) for c in range(2)]
    h1 = _tc_conv0(h0p, tx0_list, w0p[0], w0_half,
                   conv0_b.reshape(1, 128))

    h1_halves = h1.reshape(NP, 2, 64).transpose(1, 0, 2)
    tx1, _ = _sc_layer1(h1_halves, rowp, colp, dinv)

    w1_half = jnp.stack([conv1_W[k, c * 64:(c + 1) * 64]
                         for k in (1, 2, 3) for c in (0, 1)])
    tx1_list = [tx1[k, c] for k in range(3) for c in range(2)]
    mu_full, lv_full = _tc_heads(
        h1, tx1_list, conv1_W[0], w1_half, conv1_b.reshape(1, 128),
        mu_W, mu_b.reshape(1, 64), lv_W, lv_b.reshape(1, 64))

    return mu_full[:N], lv_full[:N]


# speedup vs baseline: 2.1087x; 1.0126x over previous
import jax
import jax.numpy as jnp
from jax import lax
from jax.experimental import pallas as pl
from jax.experimental.pallas import tpu as pltpu
from jax.experimental.pallas import tpu_sc as plsc

N = 10000
NP = 10240
E = 320000
NTILES = 16
PT = NP // NTILES
CR = 40
NCHN = PT // CR
CED = 80
NCH = E // (NTILES * CED)
NTRASH = 10000


def _bcast(ref, idx):
    return plsc.load_gather(ref, [jnp.zeros((16,), jnp.int32) + idx])


def _make_sc_layer(width, prep):
    mesh = plsc.VectorSubcoreMesh(core_axis_name="c", subcore_axis_name="s")

    if prep:
        out_type = (
            jax.ShapeDtypeStruct((3, 2, NP, width), jnp.float32),
            jax.ShapeDtypeStruct((2, NP, width), jnp.float32),
            jax.ShapeDtypeStruct((NP,), jnp.float32),
        )
    else:
        out_type = (
            jax.ShapeDtypeStruct((3, 2, NP, width), jnp.float32),
            jax.ShapeDtypeStruct((2, NP, width), jnp.float32),
        )

    scratch = [
        pltpu.VMEM_SHARED((NP, width), jnp.float32),
        pltpu.VMEM((4, CED), jnp.int32),
        pltpu.VMEM((4, CED), jnp.int32),
        pltpu.VMEM((4, CED, width), jnp.float32),
        pltpu.VMEM((CR, width), jnp.float32),
        pltpu.VMEM((CR, width), jnp.float32),
        pltpu.VMEM((CR, width), jnp.float32),
        pltpu.VMEM((CR, width), jnp.float32),
        pltpu.VMEM((CR, width), jnp.float32),
        pltpu.VMEM((PT,), jnp.float32),
        pltpu.VMEM((CED,), jnp.float32),
        pltpu.SemaphoreType.DMA,
        pltpu.SemaphoreType.DMA,
        pltpu.SemaphoreType.DMA,
    ]
    if prep:
        scratch.insert(1, pltpu.VMEM_SHARED((NP,), jnp.float32))

    def body(*refs):
        if prep:
            (h_hbm, row_hbm, col_hbm, tx_hbm, tp_hbm, dinv_hbm,
             acc, deg, ridx, cidx, gbuf,
             abuf, pbuf, tbuf, sbuf, zbuf, dbuf, vbuf,
             gsem, isem, ssem) = refs
        else:
            (h_hbm, row_hbm, col_hbm, dinv_in_hbm, tx_hbm, tp_hbm,
             acc, ridx, cidx, gbuf,
             abuf, pbuf, tbuf, sbuf, zbuf, dbuf, vbuf,
             gsem, isem, ssem) = refs

        cid = lax.axis_index("c")
        sid = lax.axis_index("s")
        r0 = sid * PT
        z16 = jnp.zeros((16,), jnp.float32)
        tpc = tp_hbm.at[cid]

        def zero_rows(ref, nrows):
            def zr(j, _):
                for v in range(width // 16):
                    ref[j, pl.ds(16 * v, 16)] = z16
                return 0
            lax.fori_loop(0, nrows, zr, 0)

        zero_rows(zbuf, CR)

        def fire_idx(i, b):
            pltpu.async_copy(row_hbm.at[sid, i], ridx.at[b], isem)
            pltpu.async_copy(col_hbm.at[sid, i], cidx.at[b], isem)

        def wait_idx():
            pltpu.make_async_copy(
                row_hbm.at[sid, 0], ridx.at[0], isem).wait()
            pltpu.make_async_copy(
                col_hbm.at[sid, 0], cidx.at[0], isem).wait()

        def load_idx_sync(i, b):
            pltpu.sync_copy(row_hbm.at[sid, i], ridx.at[b])
            pltpu.sync_copy(col_hbm.at[sid, i], cidx.at[b])

        def remap(b):
            for v in range(CED // 16):
                r = ridx[b, pl.ds(16 * v, 16)]
                c = cidx[b, pl.ds(16 * v, 16)]
                ridx[b, pl.ds(16 * v, 16)] = jnp.where(r == c, NTRASH, r)

        if prep:
            def zd(j, _):
                dbuf[pl.ds(16 * j, 16)] = z16
                return 0
            lax.fori_loop(0, PT // 16, zd, 0)
            pltpu.sync_copy(dbuf, deg.at[pl.ds(r0, PT)])

            one = jnp.ones((16,), jnp.float32)

            def ob(j, _):
                vbuf[pl.ds(16 * j, 16)] = one
                return 0
            lax.fori_loop(0, CED // 16, ob, 0)
            plsc.subcore_barrier()

            def dscat(b):
                pltpu.async_copy(vbuf, deg.at[ridx.at[b]], ssem, add=True)

            def wait_dscat():
                pltpu.make_async_copy(
                    vbuf, deg.at[ridx.at[0]], ssem).wait()

            load_idx_sync(0, 0)
            remap(0)
            dscat(0)
            fire_idx(1, 1)
            wait_idx()
            remap(1)
            fire_idx(2, 2)
            dscat(1)
            wait_idx()
            remap(2)
            fire_idx(3, 3)
            dscat(2)

            def dproc(i, s):
                wait_idx()
                remap(s)
                wait_dscat()
                fire_idx(i + 1, (s + 1) % 4)
                dscat(s)

            def dquad(q, _):
                i = 4 * q + 3
                dproc(i, 3)
                dproc(i + 1, 0)
                dproc(i + 2, 1)
                dproc(i + 3, 2)
                return 0

            lax.fori_loop(0, (NCH - 6) // 4, dquad, 0)
            dproc(NCH - 3, 3)
            dproc(NCH - 2, 0)
            wait_idx()
            remap(1)
            wait_dscat()
            dscat(1)
            wait_dscat()
            wait_dscat()
            wait_dscat()
            plsc.subcore_barrier()

            pltpu.sync_copy(deg.at[pl.ds(r0, PT)], dbuf)

            def dv(j, _):
                d = dbuf[pl.ds(16 * j, 16)]
                ii = lax.bitcast_convert_type(d, jnp.int32)
                yi = jnp.int32(0x5F3759DF) - (ii >> 1)
                y = lax.bitcast_convert_type(yi, jnp.float32)
                for _ in range(3):
                    y = y * (1.5 - 0.5 * d * y * y)
                dbuf[pl.ds(16 * j, 16)] = jnp.where(d > 0.5, y, 0.0)
                return 0

            lax.fori_loop(0, PT // 16, dv, 0)

            @pl.when(cid == 0)
            def _():
                pltpu.sync_copy(dbuf, dinv_hbm.at[pl.ds(r0, PT)])
        else:
            pltpu.sync_copy(dinv_in_hbm.at[pl.ds(r0, PT)], dbuf)

        def init_chunk(cc, _):
            rr = r0 + cc * CR
            pltpu.sync_copy(h_hbm.at[cid, pl.ds(rr, CR), :], pbuf)

            def tinit(j, _):
                d = _bcast(dbuf, cc * CR + j)
                for v in range(width // 16):
                    sbuf[j, pl.ds(16 * v, 16)] = d * pbuf[j, pl.ds(16 * v, 16)]
                return 0

            lax.fori_loop(0, CR, tinit, 0)
            pltpu.sync_copy(sbuf, tpc.at[pl.ds(rr, CR), :])
            pltpu.sync_copy(zbuf, acc.at[pl.ds(rr, CR), :])
            return 0

        lax.fori_loop(0, NCHN, init_chunk, 0)
        plsc.subcore_barrier()

        def fire_gather(b):
            pltpu.async_copy(tpc.at[ridx.at[b]], gbuf.at[b], gsem)

        def wait_gather():
            pltpu.make_async_copy(
                tpc.at[ridx.at[0]], gbuf.at[0], gsem).wait()

        def scatter(b):
            pltpu.sync_copy(gbuf.at[b], acc.at[cidx.at[b]], add=True)

        def fire_scatter(b):
            pltpu.async_copy(gbuf.at[b], acc.at[cidx.at[b]], ssem, add=True)

        def wait_scatter():
            pltpu.make_async_copy(
                gbuf.at[0], acc.at[cidx.at[0]], ssem).wait()

        for k in (1, 2, 3):
            load_idx_sync(0, 0)
            remap(0)
            fire_gather(0)
            fire_idx(1, 1)

            def process(i, s):
                wait_idx()
                remap((s + 1) % 4)
                fire_gather((s + 1) % 4)
                wait_scatter()
                fire_idx(i + 2, (s + 2) % 4)
                wait_gather()
                fire_scatter(s)

            wait_idx()
            remap(1)
            fire_gather(1)
            fire_idx(2, 2)
            wait_gather()
            fire_scatter(0)
            wait_idx()
            remap(2)
            fire_gather(2)
            fire_idx(3, 3)
            wait_gather()
            fire_scatter(1)

            def quad(q, _):
                i = 4 * q + 2
                process(i, 2)
                process(i + 1, 3)
                process(i + 2, 0)
                process(i + 3, 1)
                return 0

            lax.fori_loop(0, (NCH - 6) // 4, quad, 0)
            process(NCH - 4, 2)
            process(NCH - 3, 3)
            wait_idx()
            remap(1)
            fire_gather(1)
            wait_scatter()
            wait_gather()
            fire_scatter(0)
            wait_scatter()
            wait_gather()
            fire_scatter(1)
            wait_scatter()
            wait_scatter()
            plsc.subcore_barrier()

            def fix_chunk(cc, _):
                rr = r0 + cc * CR
                pltpu.sync_copy(acc.at[pl.ds(rr, CR), :], abuf)
                if k == 2:
                    pltpu.sync_copy(h_hbm.at[cid, pl.ds(rr, CR), :], pbuf)
                elif k == 3:
                    pltpu.sync_copy(
                        tx_hbm.at[0, cid, pl.ds(rr, CR), :], pbuf)

                def fix(j, _):
                    d = _bcast(dbuf, cc * CR + j)
                    for v in range(width // 16):
                        a = abuf[j, pl.ds(16 * v, 16)]
                        if k == 1:
                            t = (-d) * a
                        else:
                            t = ((-2.0 * d) * a
                                 - pbuf[j, pl.ds(16 * v, 16)])
                        tbuf[j, pl.ds(16 * v, 16)] = t
                        if k < 3:
                            sbuf[j, pl.ds(16 * v, 16)] = d * t
                    return 0

                lax.fori_loop(0, CR, fix, 0)
                pltpu.sync_copy(tbuf, tx_hbm.at[k - 1, cid, pl.ds(rr, CR), :])
                if k < 3:
                    pltpu.sync_copy(sbuf, tpc.at[pl.ds(rr, CR), :])
                    pltpu.sync_copy(zbuf, acc.at[pl.ds(rr, CR), :])
                return 0

            lax.fori_loop(0, NCHN, fix_chunk, 0)
            plsc.subcore_barrier()

    return pl.kernel(body, out_type=out_type, mesh=mesh,
                     scratch_types=scratch,
                     compiler_params=pltpu.CompilerParams(
                         needs_layout_passes=False,
                         use_tc_tiling_on_sc=False))


def _tc_conv0(h0p, txs, wfull, whalf, b0):
    RB = 1024
    grid = (NP // RB,)

    def body(h_ref, t0, t1, t2, t3, t4, t5, wf_ref, wh_ref, b_ref, o_ref):
        acc = jnp.dot(h_ref[...], wf_ref[...],
                      preferred_element_type=jnp.float32)
        for i, t in enumerate((t0, t1, t2, t3, t4, t5)):
            acc += jnp.dot(t[...], wh_ref[i],
                           preferred_element_type=jnp.float32)
        acc += b_ref[...]
        acc = jnp.maximum(acc, 0.0)
        rid = (pl.program_id(0) * RB
               + lax.broadcasted_iota(jnp.int32, (RB, 1), 0))
        o_ref[...] = jnp.where(rid < N, acc, 0.0)

    hw = h0p.shape[1]
    tw = txs[0].shape[1]
    return pl.pallas_call(
        body,
        grid=grid,
        in_specs=[pl.BlockSpec((RB, hw), lambda i: (i, 0))]
        + [pl.BlockSpec((RB, tw), lambda i: (i, 0))] * 6
        + [pl.BlockSpec(wfull.shape, lambda i: (0, 0)),
           pl.BlockSpec(whalf.shape, lambda i: (0, 0, 0)),
           pl.BlockSpec(b0.shape, lambda i: (0, 0))],
        out_specs=pl.BlockSpec((RB, 128), lambda i: (i, 0)),
        out_shape=jax.ShapeDtypeStruct((NP, 128), jnp.float32),
    )(h0p, *txs, wfull, whalf, b0)


def _tc_heads(h1, txs, wfull, whalf, b1, mu_w, mu_b, lv_w, lv_b):
    RB = 1024
    grid = (NP // RB,)

    def body(h_ref, t0, t1, t2, t3, t4, t5, wf_ref, wh_ref, b_ref,
             muw_ref, mub_ref, lvw_ref, lvb_ref, mu_ref, lv_ref):
        acc = jnp.dot(h_ref[...], wf_ref[...],
                      preferred_element_type=jnp.float32)
        for i, t in enumerate((t0, t1, t2, t3, t4, t5)):
            acc += jnp.dot(t[...], wh_ref[i],
                           preferred_element_type=jnp.float32)
        h2 = jnp.maximum(acc + b_ref[...], 0.0)
        mu_ref[...] = jnp.dot(h2, muw_ref[...],
                              preferred_element_type=jnp.float32) + mub_ref[...]
        lv_ref[...] = jnp.dot(h2, lvw_ref[...],
                              preferred_element_type=jnp.float32) + lvb_ref[...]

    return pl.pallas_call(
        body,
        grid=grid,
        in_specs=[pl.BlockSpec((RB, 128), lambda i: (i, 0))]
        + [pl.BlockSpec((RB, 64), lambda i: (i, 0))] * 6
        + [pl.BlockSpec(wfull.shape, lambda i: (0, 0)),
           pl.BlockSpec(whalf.shape, lambda i: (0, 0, 0)),
           pl.BlockSpec(b1.shape, lambda i: (0, 0)),
           pl.BlockSpec(mu_w.shape, lambda i: (0, 0)),
           pl.BlockSpec(mu_b.shape, lambda i: (0, 0)),
           pl.BlockSpec(lv_w.shape, lambda i: (0, 0)),
           pl.BlockSpec(lv_b.shape, lambda i: (0, 0))],
        out_specs=[pl.BlockSpec((RB, 64), lambda i: (i, 0)),
                   pl.BlockSpec((RB, 64), lambda i: (i, 0))],
        out_shape=[jax.ShapeDtypeStruct((NP, 64), jnp.float32),
                   jax.ShapeDtypeStruct((NP, 64), jnp.float32)],
    )(h1, *txs, wfull, whalf, b1, mu_w, mu_b, lv_w, lv_b)


_sc_layer0 = _make_sc_layer(80, prep=True)
_sc_layer1 = _make_sc_layer(64, prep=False)


def kernel(x, edge_index, lap_pe, conv0_W, conv0_b, conv1_W, conv1_b,
           mu_W, mu_b, lv_W, lv_b):
    f32 = jnp.float32
    h0 = jnp.concatenate([x, lap_pe], axis=1)
    h0p = jnp.zeros((NP, 160), f32).at[:N, :144].set(h0)
    h0_halves = h0p.reshape(NP, 2, 80).transpose(1, 0, 2)

    rowp = edge_index[0].reshape(NTILES, NCH, CED)
    colp = edge_index[1].reshape(NTILES, NCH, CED)

    tx0, _, dinv = _sc_layer0(h0_halves, rowp, colp)

    w0p = jnp.zeros((4, 160, 128), f32).at[:, :144, :].set(conv0_W)
    w0_half = jnp.stack([w0p[k, c * 80:(c + 1) * 80]
                         for k in (1, 2, 3) for c in (0, 1)])
    tx0_list = [tx0[k, c] for k in range(3) for c in range(2)]
    h1 = _tc_conv0(h0p, tx0_list, w0p[0], w0_half,
                   conv0_b.reshape(1, 128))

    h1_halves = h1.reshape(NP, 2, 64).transpose(1, 0, 2)
    tx1, _ = _sc_layer1(h1_halves, rowp, colp, dinv)

    w1_half = jnp.stack([conv1_W[k, c * 64:(c + 1) * 64]
                         for k in (1, 2, 3) for c in (0, 1)])
    tx1_list = [tx1[k, c] for k in range(3) for c in range(2)]
    mu_full, lv_full = _tc_heads(
        h1, tx1_list, conv1_W[0], w1_half, conv1_b.reshape(1, 128),
        mu_W, mu_b.reshape(1, 64), lv_W, lv_b.reshape(1, 64))

    return mu_full[:N], lv_full[:N]


# speedup vs baseline: 2.1554x; 1.0222x over previous
import jax
import jax.numpy as jnp
from jax import lax
from jax.experimental import pallas as pl
from jax.experimental.pallas import tpu as pltpu
from jax.experimental.pallas import tpu_sc as plsc

N = 10000
NP = 10240
E = 320000
NTILES = 16
PT = NP // NTILES
CR = 40
NCHN = PT // CR
CED = 80
NCH = E // (NTILES * CED)
NTRASH = 10000


def _bcast(ref, idx):
    return plsc.load_gather(ref, [jnp.zeros((16,), jnp.int32) + idx])


def _make_sc_layer(width, prep):
    mesh = plsc.VectorSubcoreMesh(core_axis_name="c", subcore_axis_name="s")

    if prep:
        out_type = (
            jax.ShapeDtypeStruct((3, 2, NP, width), jnp.float32),
            jax.ShapeDtypeStruct((2, NP, width), jnp.float32),
            jax.ShapeDtypeStruct((NP,), jnp.float32),
        )
    else:
        out_type = (
            jax.ShapeDtypeStruct((3, 2, NP, width), jnp.float32),
            jax.ShapeDtypeStruct((2, NP, width), jnp.float32),
        )

    scratch = [
        pltpu.VMEM_SHARED((NP, width), jnp.float32),
        pltpu.VMEM((4, CED), jnp.int32),
        pltpu.VMEM((4, CED), jnp.int32),
        pltpu.VMEM((4, CED, width), jnp.float32),
        pltpu.VMEM((CR, width), jnp.float32),
        pltpu.VMEM((CR, width), jnp.float32),
        pltpu.VMEM((CR, width), jnp.float32),
        pltpu.VMEM((CR, width), jnp.float32),
        pltpu.VMEM((CR, width), jnp.float32),
        pltpu.VMEM((PT,), jnp.float32),
        pltpu.VMEM((CED,), jnp.float32),
        pltpu.SemaphoreType.DMA,
        pltpu.SemaphoreType.DMA,
        pltpu.SemaphoreType.DMA,
    ]
    if prep:
        scratch.insert(1, pltpu.VMEM_SHARED((NP,), jnp.float32))

    def body(*refs):
        if prep:
            (h_hbm, row_hbm, col_hbm, tx_hbm, tp_hbm, dinv_hbm,
             acc, deg, ridx, cidx, gbuf,
             abuf, pbuf, tbuf, sbuf, zbuf, dbuf, vbuf,
             gsem, isem, ssem) = refs
        else:
            (h_hbm, row_hbm, col_hbm, dinv_in_hbm, tx_hbm, tp_hbm,
             acc, ridx, cidx, gbuf,
             abuf, pbuf, tbuf, sbuf, zbuf, dbuf, vbuf,
             gsem, isem, ssem) = refs

        cid = lax.axis_index("c")
        sid = lax.axis_index("s")
        r0 = sid * PT
        z16 = jnp.zeros((16,), jnp.float32)
        tpc = tp_hbm.at[cid]

        def zero_rows(ref, nrows):
            def zr(j, _):
                for v in range(width // 16):
                    ref[j, pl.ds(16 * v, 16)] = z16
                return 0
            lax.fori_loop(0, nrows, zr, 0)

        zero_rows(zbuf, CR)

        def fire_idx(i, b):
            pltpu.async_copy(row_hbm.at[sid, i], ridx.at[b], isem)
            pltpu.async_copy(col_hbm.at[sid, i], cidx.at[b], isem)

        def wait_idx():
            pltpu.make_async_copy(
                row_hbm.at[sid, 0], ridx.at[0], isem).wait()
            pltpu.make_async_copy(
                col_hbm.at[sid, 0], cidx.at[0], isem).wait()

        def load_idx_sync(i, b):
            pltpu.sync_copy(row_hbm.at[sid, i], ridx.at[b])
            pltpu.sync_copy(col_hbm.at[sid, i], cidx.at[b])

        def remap(b):
            for v in range(CED // 16):
                r = ridx[b, pl.ds(16 * v, 16)]
                c = cidx[b, pl.ds(16 * v, 16)]
                ridx[b, pl.ds(16 * v, 16)] = jnp.where(r == c, NTRASH, r)

        if prep:
            def zd(j, _):
                dbuf[pl.ds(16 * j, 16)] = z16
                return 0
            lax.fori_loop(0, PT // 16, zd, 0)
            pltpu.sync_copy(dbuf, deg.at[pl.ds(r0, PT)])

            one = jnp.ones((16,), jnp.float32)

            def ob(j, _):
                vbuf[pl.ds(16 * j, 16)] = one
                return 0
            lax.fori_loop(0, CED // 16, ob, 0)
            plsc.subcore_barrier()

            def dscat(b):
                pltpu.async_copy(vbuf, deg.at[ridx.at[b]], ssem, add=True)

            def wait_dscat():
                pltpu.make_async_copy(
                    vbuf, deg.at[ridx.at[0]], ssem).wait()

            load_idx_sync(0, 0)
            remap(0)
            dscat(0)
            fire_idx(1, 1)
            wait_idx()
            remap(1)
            fire_idx(2, 2)
            dscat(1)
            wait_idx()
            remap(2)
            fire_idx(3, 3)
            dscat(2)

            def dproc(i, s):
                wait_idx()
                remap(s)
                wait_dscat()
                fire_idx(i + 1, (s + 1) % 4)
                dscat(s)

            def dquad(q, _):
                i = 4 * q + 3
                dproc(i, 3)
                dproc(i + 1, 0)
                dproc(i + 2, 1)
                dproc(i + 3, 2)
                return 0

            lax.fori_loop(0, (NCH - 6) // 4, dquad, 0)
            dproc(NCH - 3, 3)
            dproc(NCH - 2, 0)
            wait_idx()
            remap(1)
            wait_dscat()
            dscat(1)
            wait_dscat()
            wait_dscat()
            wait_dscat()
            plsc.subcore_barrier()

            pltpu.sync_copy(deg.at[pl.ds(r0, PT)], dbuf)

            def dv(j, _):
                d = dbuf[pl.ds(16 * j, 16)]
                ii = lax.bitcast_convert_type(d, jnp.int32)
                yi = jnp.int32(0x5F3759DF) - (ii >> 1)
                y = lax.bitcast_convert_type(yi, jnp.float32)
                for _ in range(3):
                    y = y * (1.5 - 0.5 * d * y * y)
                dbuf[pl.ds(16 * j, 16)] = jnp.where(d > 0.5, y, 0.0)
                return 0

            lax.fori_loop(0, PT // 16, dv, 0)

            @pl.when(cid == 0)
            def _():
                pltpu.sync_copy(dbuf, dinv_hbm.at[pl.ds(r0, PT)])
        else:
            pltpu.sync_copy(dinv_in_hbm.at[pl.ds(r0, PT)], dbuf)

        def init_chunk(cc, _):
            rr = r0 + cc * CR
            pltpu.sync_copy(
                h_hbm.at[pl.ds(rr, CR), pl.ds(cid * width, width)], pbuf)

            def tinit(j, _):
                d = _bcast(dbuf, cc * CR + j)
                for v in range(width // 16):
                    sbuf[j, pl.ds(16 * v, 16)] = d * pbuf[j, pl.ds(16 * v, 16)]
                return 0

            lax.fori_loop(0, CR, tinit, 0)
            pltpu.sync_copy(sbuf, tpc.at[pl.ds(rr, CR), :])
            pltpu.sync_copy(zbuf, acc.at[pl.ds(rr, CR), :])
            return 0

        lax.fori_loop(0, NCHN, init_chunk, 0)
        plsc.subcore_barrier()

        def fire_gather(b):
            pltpu.async_copy(tpc.at[ridx.at[b]], gbuf.at[b], gsem)

        def wait_gather():
            pltpu.make_async_copy(
                tpc.at[ridx.at[0]], gbuf.at[0], gsem).wait()

        def scatter(b):
            pltpu.sync_copy(gbuf.at[b], acc.at[cidx.at[b]], add=True)

        def fire_scatter(b):
            pltpu.async_copy(gbuf.at[b], acc.at[cidx.at[b]], ssem, add=True)

        def wait_scatter():
            pltpu.make_async_copy(
                gbuf.at[0], acc.at[cidx.at[0]], ssem).wait()

        for k in (1, 2, 3):
            load_idx_sync(0, 0)
            remap(0)
            fire_gather(0)
            fire_idx(1, 1)

            def process(i, s):
                wait_idx()
                remap((s + 1) % 4)
                fire_gather((s + 1) % 4)
                wait_scatter()
                fire_idx(i + 2, (s + 2) % 4)
                wait_gather()
                fire_scatter(s)

            wait_idx()
            remap(1)
            fire_gather(1)
            fire_idx(2, 2)
            wait_gather()
            fire_scatter(0)
            wait_idx()
            remap(2)
            fire_gather(2)
            fire_idx(3, 3)
            wait_gather()
            fire_scatter(1)

            def quad(q, _):
                i = 4 * q + 2
                process(i, 2)
                process(i + 1, 3)
                process(i + 2, 0)
                process(i + 3, 1)
                return 0

            lax.fori_loop(0, (NCH - 6) // 4, quad, 0)
            process(NCH - 4, 2)
            process(NCH - 3, 3)
            wait_idx()
            remap(1)
            fire_gather(1)
            wait_scatter()
            wait_gather()
            fire_scatter(0)
            wait_scatter()
            wait_gather()
            fire_scatter(1)
            wait_scatter()
            wait_scatter()
            plsc.subcore_barrier()

            def fix_chunk(cc, _):
                rr = r0 + cc * CR
                pltpu.sync_copy(acc.at[pl.ds(rr, CR), :], abuf)
                if k == 2:
                    pltpu.sync_copy(
                h_hbm.at[pl.ds(rr, CR), pl.ds(cid * width, width)], pbuf)
                elif k == 3:
                    pltpu.sync_copy(
                        tx_hbm.at[0, cid, pl.ds(rr, CR), :], pbuf)

                def fix(j, _):
                    d = _bcast(dbuf, cc * CR + j)
                    for v in range(width // 16):
                        a = abuf[j, pl.ds(16 * v, 16)]
                        if k == 1:
                            t = (-d) * a
                        else:
                            t = ((-2.0 * d) * a
                                 - pbuf[j, pl.ds(16 * v, 16)])
                        tbuf[j, pl.ds(16 * v, 16)] = t
                        if k < 3:
                            sbuf[j, pl.ds(16 * v, 16)] = d * t
                    return 0

                lax.fori_loop(0, CR, fix, 0)
                pltpu.sync_copy(tbuf, tx_hbm.at[k - 1, cid, pl.ds(rr, CR), :])
                if k < 3:
                    pltpu.sync_copy(sbuf, tpc.at[pl.ds(rr, CR), :])
                    pltpu.sync_copy(zbuf, acc.at[pl.ds(rr, CR), :])
                return 0

            lax.fori_loop(0, NCHN, fix_chunk, 0)
            plsc.subcore_barrier()

    return pl.kernel(body, out_type=out_type, mesh=mesh,
                     scratch_types=scratch,
                     compiler_params=pltpu.CompilerParams(
                         needs_layout_passes=False,
                         use_tc_tiling_on_sc=False))


def _tc_conv0(h0p, txs, wfull, whalf, b0):
    RB = 1024
    grid = (NP // RB,)

    def body(h_ref, t0, t1, t2, t3, t4, t5, wf_ref, wh_ref, b_ref, o_ref):
        acc = jnp.dot(h_ref[...], wf_ref[...],
                      preferred_element_type=jnp.float32)
        for i, t in enumerate((t0, t1, t2, t3, t4, t5)):
            acc += jnp.dot(t[...], wh_ref[i],
                           preferred_element_type=jnp.float32)
        acc += b_ref[...]
        acc = jnp.maximum(acc, 0.0)
        rid = (pl.program_id(0) * RB
               + lax.broadcasted_iota(jnp.int32, (RB, 1), 0))
        o_ref[...] = jnp.where(rid < N, acc, 0.0)

    hw = h0p.shape[1]
    tw = txs[0].shape[1]
    return pl.pallas_call(
        body,
        grid=grid,
        in_specs=[pl.BlockSpec((RB, hw), lambda i: (i, 0))]
        + [pl.BlockSpec((RB, tw), lambda i: (i, 0))] * 6
        + [pl.BlockSpec(wfull.shape, lambda i: (0, 0)),
           pl.BlockSpec(whalf.shape, lambda i: (0, 0, 0)),
           pl.BlockSpec(b0.shape, lambda i: (0, 0))],
        out_specs=pl.BlockSpec((RB, 128), lambda i: (i, 0)),
        out_shape=jax.ShapeDtypeStruct((NP, 128), jnp.float32),
    )(h0p, *txs, wfull, whalf, b0)


def _tc_heads(h1, txs, wfull, whalf, b1, mu_w, mu_b, lv_w, lv_b):
    RB = 1024
    grid = (NP // RB,)

    def body(h_ref, t0, t1, t2, t3, t4, t5, wf_ref, wh_ref, b_ref,
             muw_ref, mub_ref, lvw_ref, lvb_ref, mu_ref, lv_ref):
        acc = jnp.dot(h_ref[...], wf_ref[...],
                      preferred_element_type=jnp.float32)
        for i, t in enumerate((t0, t1, t2, t3, t4, t5)):
            acc += jnp.dot(t[...], wh_ref[i],
                           preferred_element_type=jnp.float32)
        h2 = jnp.maximum(acc + b_ref[...], 0.0)
        mu_ref[...] = jnp.dot(h2, muw_ref[...],
                              preferred_element_type=jnp.float32) + mub_ref[...]
        lv_ref[...] = jnp.dot(h2, lvw_ref[...],
                              preferred_element_type=jnp.float32) + lvb_ref[...]

    return pl.pallas_call(
        body,
        grid=grid,
        in_specs=[pl.BlockSpec((RB, 128), lambda i: (i, 0))]
        + [pl.BlockSpec((RB, 64), lambda i: (i, 0))] * 6
        + [pl.BlockSpec(wfull.shape, lambda i: (0, 0)),
           pl.BlockSpec(whalf.shape, lambda i: (0, 0, 0)),
           pl.BlockSpec(b1.shape, lambda i: (0, 0)),
           pl.BlockSpec(mu_w.shape, lambda i: (0, 0)),
           pl.BlockSpec(mu_b.shape, lambda i: (0, 0)),
           pl.BlockSpec(lv_w.shape, lambda i: (0, 0)),
           pl.BlockSpec(lv_b.shape, lambda i: (0, 0))],
        out_specs=[pl.BlockSpec((RB, 64), lambda i: (i, 0)),
                   pl.BlockSpec((RB, 64), lambda i: (i, 0))],
        out_shape=[jax.ShapeDtypeStruct((NP, 64), jnp.float32),
                   jax.ShapeDtypeStruct((NP, 64), jnp.float32)],
    )(h1, *txs, wfull, whalf, b1, mu_w, mu_b, lv_w, lv_b)


_sc_layer0 = _make_sc_layer(80, prep=True)
_sc_layer1 = _make_sc_layer(64, prep=False)


def kernel(x, edge_index, lap_pe, conv0_W, conv0_b, conv1_W, conv1_b,
           mu_W, mu_b, lv_W, lv_b):
    f32 = jnp.float32
    h0 = jnp.concatenate([x, lap_pe], axis=1)
    h0p = jnp.zeros((NP, 160), f32).at[:N, :144].set(h0)

    rowp = edge_index[0].reshape(NTILES, NCH, CED)
    colp = edge_index[1].reshape(NTILES, NCH, CED)

    tx0, _, dinv = _sc_layer0(h0p, rowp, colp)

    w0p = jnp.zeros((4, 160, 128), f32).at[:, :144, :].set(conv0_W)
    w0_half = jnp.stack([w0p[k, c * 80:(c + 1) * 80]
                         for k in (1, 2, 3) for c in (0, 1)])
    tx0_list = [tx0[k, c] for k in range(3) for c in range(2)]
    h1 = _tc_conv0(h0p, tx0_list, w0p[0], w0_half,
                   conv0_b.reshape(1, 128))

    tx1, _ = _sc_layer1(h1, rowp, colp, dinv)

    w1_half = jnp.stack([conv1_W[k, c * 64:(c + 1) * 64]
                         for k in (1, 2, 3) for c in (0, 1)])
    tx1_list = [tx1[k, c] for k in range(3) for c in range(2)]
    mu_full, lv_full = _tc_heads(
        h1, tx1_list, conv1_W[0], w1_half, conv1_b.reshape(1, 128),
        mu_W, mu_b.reshape(1, 64), lv_W, lv_b.reshape(1, 64))

    return mu_full[:N], lv_full[:N]
